# Initial kernel scaffold; baseline (speedup 1.0000x reference)
import jax
import jax.numpy as jnp
from jax.experimental import pallas as pl


def kernel(x, edge_index, y, h_prev, y_prev, step_count, params):
    raise NotImplementedError("write your pallas kernel here")


# speedup vs baseline: 12.1016x; 12.1016x over previous
import functools

import numpy as np
import jax
import jax.numpy as jnp
from jax import lax
from jax.experimental import pallas as pl
from jax.experimental.pallas import tpu as pltpu
from jax.experimental.pallas import tpu_sc as plsc

N = 10000
E = 320000
H = 128
NPAD = 10240
ROWS_PER_SUB = NPAD // 16
EDGES_PER_SUB = E // 32
CHUNKS = 80
CHUNK = 125

_INV_SQRT2 = float(1.0 / np.sqrt(2.0))

_DOT = functools.partial(
    jnp.dot, precision=lax.Precision.HIGHEST,
    preferred_element_type=jnp.float32)


def _ln(v, g, b):
    mu = jnp.mean(v, axis=-1, keepdims=True)
    var = jnp.mean((v - mu) ** 2, axis=-1, keepdims=True)
    return (v - mu) * lax.rsqrt(var + 1e-5) * g + b


def _gelu(v):
    return 0.5 * v * (1.0 + lax.erf(v * _INV_SQRT2))


def _input_body(x_r, hp_r, yp_r, xew_r, xeb_r, xng_r, xnb_r,
                ipa_r, ipb_r, ipc_r, ipbias_r, ing_r, inb_r, o_r):
    x_emb = _ln(_DOT(x_r[...], xew_r[...]) + xeb_r[...], xng_r[...], xnb_r[...])
    y_in = jax.nn.sigmoid(yp_r[...])
    pre = (_DOT(x_emb, ipa_r[...]) + _DOT(hp_r[...], ipb_r[...])
           + y_in * ipc_r[...] + ipbias_r[...])
    o_r[...] = _ln(pre, ing_r[...], inb_r[...])


def _tc_input(x, h_prev, y_prev, xew, xeb, xng, xnb, ipa, ipb, ipc, ipbias,
              ing, inb):
    blk = 1000
    grid = (N // blk,)
    row = lambda i: (i, 0)
    full = lambda i: (0, 0)
    return pl.pallas_call(
        _input_body,
        grid=grid,
        in_specs=[
            pl.BlockSpec((blk, H), row),
            pl.BlockSpec((blk, H), row),
            pl.BlockSpec((blk, 1), row),
            pl.BlockSpec((H, H), full),
            pl.BlockSpec((1, H), full),
            pl.BlockSpec((1, H), full),
            pl.BlockSpec((1, H), full),
            pl.BlockSpec((H, H), full),
            pl.BlockSpec((H, H), full),
            pl.BlockSpec((1, H), full),
            pl.BlockSpec((1, H), full),
            pl.BlockSpec((1, H), full),
            pl.BlockSpec((1, H), full),
        ],
        out_specs=pl.BlockSpec((blk, H), row),
        out_shape=jax.ShapeDtypeStruct((N, H), jnp.float32),
    )(x, h_prev, y_prev, xew, xeb, xng, xnb, ipa, ipb, ipc, ipbias, ing, inb)


def _layer_body(h_r, p0_r, p1_r, eps_r, m1w_r, m1b_r, mng_r, mnb_r,
                m2w_r, m2b_r, ng_r, nb_r, o_r):
    h = h_r[...]
    zc = (1.0 + eps_r[...]) * h + p0_r[...] + p1_r[...]
    m = _DOT(zc, m1w_r[...]) + m1b_r[...]
    m = _gelu(_ln(m, mng_r[...], mnb_r[...]))
    m = _DOT(m, m2w_r[...]) + m2b_r[...]
    o_r[...] = _ln(h + _gelu(m), ng_r[...], nb_r[...])


def _tc_layer(h, p0, p1, eps, m1w, m1b, mng, mnb, m2w, m2b, ng, nb):
    blk = 1000
    grid = (N // blk,)
    row = lambda i: (i, 0)
    full = lambda i: (0, 0)
    return pl.pallas_call(
        _layer_body,
        grid=grid,
        in_specs=[
            pl.BlockSpec((blk, H), row),
            pl.BlockSpec((blk, H), row),
            pl.BlockSpec((blk, H), row),
            pl.BlockSpec((1, 1), full),
            pl.BlockSpec((H, 2 * H), full),
            pl.BlockSpec((1, 2 * H), full),
            pl.BlockSpec((1, 2 * H), full),
            pl.BlockSpec((1, 2 * H), full),
            pl.BlockSpec((2 * H, H), full),
            pl.BlockSpec((1, H), full),
            pl.BlockSpec((1, H), full),
            pl.BlockSpec((1, H), full),
        ],
        out_specs=pl.BlockSpec((blk, H), row),
        out_shape=jax.ShapeDtypeStruct((N, H), jnp.float32),
    )(h, p0, p1, eps, m1w, m1b, mng, mnb, m2w, m2b, ng, nb)


def _head_body(h_r, yf_r, h1w_r, h1b_r, h2w_r, h2b_r,
               logits_r, probs_r, bce_r):
    t = _gelu(_DOT(h_r[...], h1w_r[...]) + h1b_r[...])
    lg = jnp.sum(t * h2w_r[...], axis=-1, keepdims=True) + h2b_r[...]
    logits_r[...] = lg
    z = jnp.clip(lg, -10.0, 10.0)
    probs_r[...] = jax.nn.sigmoid(z)
    labels = yf_r[...]
    pos = jnp.clip(jnp.sum(labels), 1.0, None)
    neg = jnp.clip(float(N) - pos, 1.0, None)
    pw = neg / pos
    ls_pos = -jnp.log1p(jnp.exp(-z))
    ls_neg = -jnp.log1p(jnp.exp(z))
    terms = -(pw * labels * ls_pos + (1.0 - labels) * ls_neg)
    bce_r[...] = jnp.reshape(jnp.mean(terms), (1, 1))


def _tc_head(h, yf, h1w, h1b, h2w, h2b):
    full = lambda: (0, 0)
    return pl.pallas_call(
        _head_body,
        in_specs=[
            pl.BlockSpec((N, H), full),
            pl.BlockSpec((N, 1), full),
            pl.BlockSpec((H, H), full),
            pl.BlockSpec((1, H), full),
            pl.BlockSpec((1, H), full),
            pl.BlockSpec((1, 1), full),
        ],
        out_specs=[
            pl.BlockSpec((N, 1), full),
            pl.BlockSpec((N, 1), full),
            pl.BlockSpec((1, 1), full),
        ],
        out_shape=[
            jax.ShapeDtypeStruct((N, 1), jnp.float32),
            jax.ShapeDtypeStruct((N, 1), jnp.float32),
            jax.ShapeDtypeStruct((1, 1), jnp.float32),
        ],
    )(h, yf, h1w, h1b, h2w, h2b)


def _combine_body(bce_r, fp_r, loss_r):
    loss_r[...] = bce_r[...] + 50.0 * (jnp.sum(fp_r[...]) / float(E))


def _tc_combine(bce, fparts):
    full = lambda: (0, 0)
    return pl.pallas_call(
        _combine_body,
        in_specs=[
            pl.BlockSpec((1, 1), full),
            pl.BlockSpec((32, 16), full),
        ],
        out_specs=pl.BlockSpec((1, 1), full),
        out_shape=jax.ShapeDtypeStruct((1, 1), jnp.float32),
    )(bce, fparts)


_MESH = plsc.VectorSubcoreMesh(core_axis_name="c", subcore_axis_name="s")


@functools.partial(
    pl.kernel,
    mesh=_MESH,
    out_type=jax.ShapeDtypeStruct((2, NPAD, H), jnp.float32),
    scratch_types=[
        pltpu.VMEM((CHUNKS, CHUNK), jnp.int32),
        pltpu.VMEM((CHUNKS, CHUNK), jnp.int32),
        pltpu.VMEM((CHUNK, H), jnp.float32),
        pltpu.VMEM_SHARED((NPAD, H), jnp.float32),
    ],
)
def _sc_segsum(h_hbm, ei_hbm, z_hbm, out_hbm, src_v, dst_v, rows_v, agg_sh):
    c = lax.axis_index("c")
    s = lax.axis_index("s")
    base = s * ROWS_PER_SUB
    pltpu.sync_copy(z_hbm.at[pl.ds(base, ROWS_PER_SUB)],
                    agg_sh.at[pl.ds(base, ROWS_PER_SUB)])
    pltpu.sync_copy(ei_hbm.at[0, c, s], src_v)
    pltpu.sync_copy(ei_hbm.at[1, c, s], dst_v)
    plsc.subcore_barrier()

    @pl.loop(0, CHUNKS)
    def _(j):
        pltpu.sync_copy(h_hbm.at[src_v.at[j]], rows_v)
        pltpu.sync_copy(rows_v, agg_sh.at[dst_v.at[j]], add=True)

    plsc.subcore_barrier()
    pltpu.sync_copy(agg_sh.at[pl.ds(base, ROWS_PER_SUB)],
                    out_hbm.at[c, pl.ds(base, ROWS_PER_SUB)])


@functools.partial(
    pl.kernel,
    mesh=_MESH,
    out_type=jax.ShapeDtypeStruct((32, 16), jnp.float32),
    compiler_params=pltpu.CompilerParams(needs_layout_passes=False),
    scratch_types=[
        pltpu.VMEM((N,), jnp.float32),
        pltpu.VMEM((EDGES_PER_SUB,), jnp.int32),
        pltpu.VMEM((EDGES_PER_SUB,), jnp.int32),
        pltpu.VMEM((16,), jnp.float32),
    ],
)
def _sc_feas(probs_hbm, ei_hbm, out_hbm, ptab, src_v, dst_v, acc_v):
    c = lax.axis_index("c")
    s = lax.axis_index("s")
    w = c * 16 + s
    pltpu.sync_copy(probs_hbm, ptab)
    pltpu.sync_copy(ei_hbm.at[0, w], src_v)
    pltpu.sync_copy(ei_hbm.at[1, w], dst_v)
    acc_v[...] = jnp.zeros((16,), jnp.float32)

    @pl.loop(0, EDGES_PER_SUB, step=16)
    def _(i):
        si = src_v[pl.ds(i, 16)]
        di = dst_v[pl.ds(i, 16)]
        a = plsc.load_gather(ptab, [si])
        b = plsc.load_gather(ptab, [di])
        acc_v[...] = acc_v[...] + a * b

    pltpu.sync_copy(acc_v, out_hbm.at[w])


def kernel(x, edge_index, y, h_prev, y_prev, step_count, params):
    p = params
    ei_seg = edge_index.reshape(2, 2, 16, CHUNKS, CHUNK)
    ei_feas = edge_index.reshape(2, 32, EDGES_PER_SUB)
    zeros_pad = jnp.zeros((NPAD, H), jnp.float32)

    ipa = p['ip_W'][:H]
    ipb = p['ip_W'][H:2 * H]
    ipc = p['ip_W'][2 * H:2 * H + 1]
    h = _tc_input(
        x, h_prev, y_prev,
        p['xe_W'], p['xe_b'].reshape(1, H),
        p['xn_g'].reshape(1, H), p['xn_b'].reshape(1, H),
        ipa, ipb, ipc, p['ip_b'].reshape(1, H),
        p['in_g'].reshape(1, H), p['in_b'].reshape(1, H))

    for lp in p['layers']:
        parts = _sc_segsum(h, ei_seg, zeros_pad)
        h = _tc_layer(
            h, parts[0], parts[1], lp['eps'].reshape(1, 1),
            lp['m1_W'], lp['m1_b'].reshape(1, 2 * H),
            lp['mn_g'].reshape(1, 2 * H), lp['mn_b'].reshape(1, 2 * H),
            lp['m2_W'], lp['m2_b'].reshape(1, H),
            lp['n_g'].reshape(1, H), lp['n_b'].reshape(1, H))

    yf = y.astype(jnp.float32).reshape(N, 1)
    logits, probs2d, bce = _tc_head(
        h, yf, p['h1_W'], p['h1_b'].reshape(1, H),
        p['h2_W'].reshape(1, H), p['h2_b'].reshape(1, 1))

    fparts = _sc_feas(probs2d.reshape(N), ei_feas)
    loss = _tc_combine(bce, fparts).reshape(())

    return loss, logits, h, probs2d.reshape(N)


# speedup vs baseline: 13.4598x; 1.1122x over previous
import functools

import numpy as np
import jax
import jax.numpy as jnp
from jax import lax
from jax.experimental import pallas as pl
from jax.experimental.pallas import tpu as pltpu
from jax.experimental.pallas import tpu_sc as plsc

N = 10000
E = 320000
H = 128
NPAD = 10240
ROWS_PER_SUB = NPAD // 16
EDGES_PER_SUB = E // 32
HH = H // 2
EDGES_PER_TILE = E // 16
SCHUNKS = 160
CHUNK = 125

_INV_SQRT2 = float(1.0 / np.sqrt(2.0))

_DOT = functools.partial(
    jnp.dot, precision=lax.Precision.HIGHEST,
    preferred_element_type=jnp.float32)


def _ln(v, g, b):
    mu = jnp.mean(v, axis=-1, keepdims=True)
    var = jnp.mean((v - mu) ** 2, axis=-1, keepdims=True)
    return (v - mu) * lax.rsqrt(var + 1e-5) * g + b


def _gelu(v):
    return 0.5 * v * (1.0 + lax.erf(v * _INV_SQRT2))


def _input_body(x_r, hp_r, yp_r, xew_r, xeb_r, xng_r, xnb_r,
                ipa_r, ipb_r, ipc_r, ipbias_r, ing_r, inb_r, o_r, o2_r):
    x_emb = _ln(_DOT(x_r[...], xew_r[...]) + xeb_r[...], xng_r[...], xnb_r[...])
    y_in = jax.nn.sigmoid(yp_r[...])
    pre = (_DOT(x_emb, ipa_r[...]) + _DOT(hp_r[...], ipb_r[...])
           + y_in * ipc_r[...] + ipbias_r[...])
    res = _ln(pre, ing_r[...], inb_r[...])
    o_r[...] = res
    o2_r[...] = jnp.stack((res[:, :HH], res[:, HH:]), axis=0)


def _tc_input(x, h_prev, y_prev, xew, xeb, xng, xnb, ipa, ipb, ipc, ipbias,
              ing, inb):
    blk = 1000
    grid = (N // blk,)
    row = lambda i: (i, 0)
    full = lambda i: (0, 0)
    return pl.pallas_call(
        _input_body,
        grid=grid,
        in_specs=[
            pl.BlockSpec((blk, H), row),
            pl.BlockSpec((blk, H), row),
            pl.BlockSpec((blk, 1), row),
            pl.BlockSpec((H, H), full),
            pl.BlockSpec((1, H), full),
            pl.BlockSpec((1, H), full),
            pl.BlockSpec((1, H), full),
            pl.BlockSpec((H, H), full),
            pl.BlockSpec((H, H), full),
            pl.BlockSpec((1, H), full),
            pl.BlockSpec((1, H), full),
            pl.BlockSpec((1, H), full),
            pl.BlockSpec((1, H), full),
        ],
        out_specs=[
            pl.BlockSpec((blk, H), row),
            pl.BlockSpec((2, blk, HH), lambda i: (0, i, 0)),
        ],
        out_shape=[
            jax.ShapeDtypeStruct((N, H), jnp.float32),
            jax.ShapeDtypeStruct((2, N, HH), jnp.float32),
        ],
    )(x, h_prev, y_prev, xew, xeb, xng, xnb, ipa, ipb, ipc, ipbias, ing, inb)


def _layer_body(h_r, p_r, eps_r, m1w_r, m1b_r, mng_r, mnb_r,
                m2w_r, m2b_r, ng_r, nb_r, o_r, o2_r):
    h = h_r[...]
    parts = p_r[...]
    agg = jnp.concatenate((parts[0], parts[1]), axis=-1)
    zc = (1.0 + eps_r[...]) * h + agg
    m = _DOT(zc, m1w_r[...]) + m1b_r[...]
    m = _gelu(_ln(m, mng_r[...], mnb_r[...]))
    m = _DOT(m, m2w_r[...]) + m2b_r[...]
    res = _ln(h + _gelu(m), ng_r[...], nb_r[...])
    o_r[...] = res
    o2_r[...] = jnp.stack((res[:, :HH], res[:, HH:]), axis=0)


def _tc_layer(h, parts, eps, m1w, m1b, mng, mnb, m2w, m2b, ng, nb):
    blk = 1000
    grid = (N // blk,)
    row = lambda i: (i, 0)
    full = lambda i: (0, 0)
    return pl.pallas_call(
        _layer_body,
        grid=grid,
        in_specs=[
            pl.BlockSpec((blk, H), row),
            pl.BlockSpec((2, blk, HH), lambda i: (0, i, 0)),
            pl.BlockSpec((1, 1), full),
            pl.BlockSpec((H, 2 * H), full),
            pl.BlockSpec((1, 2 * H), full),
            pl.BlockSpec((1, 2 * H), full),
            pl.BlockSpec((1, 2 * H), full),
            pl.BlockSpec((2 * H, H), full),
            pl.BlockSpec((1, H), full),
            pl.BlockSpec((1, H), full),
            pl.BlockSpec((1, H), full),
        ],
        out_specs=[
            pl.BlockSpec((blk, H), row),
            pl.BlockSpec((2, blk, HH), lambda i: (0, i, 0)),
        ],
        out_shape=[
            jax.ShapeDtypeStruct((N, H), jnp.float32),
            jax.ShapeDtypeStruct((2, N, HH), jnp.float32),
        ],
    )(h, parts, eps, m1w, m1b, mng, mnb, m2w, m2b, ng, nb)


def _head_body(h_r, yf_r, h1w_r, h1b_r, h2w_r, h2b_r,
               logits_r, probs_r, bce_r):
    t = _gelu(_DOT(h_r[...], h1w_r[...]) + h1b_r[...])
    lg = jnp.sum(t * h2w_r[...], axis=-1, keepdims=True) + h2b_r[...]
    logits_r[...] = lg
    z = jnp.clip(lg, -10.0, 10.0)
    probs_r[...] = jax.nn.sigmoid(z)
    labels = yf_r[...]
    pos = jnp.clip(jnp.sum(labels), 1.0, None)
    neg = jnp.clip(float(N) - pos, 1.0, None)
    pw = neg / pos
    ls_pos = -jnp.log1p(jnp.exp(-z))
    ls_neg = -jnp.log1p(jnp.exp(z))
    terms = -(pw * labels * ls_pos + (1.0 - labels) * ls_neg)
    bce_r[...] = jnp.reshape(jnp.mean(terms), (1, 1))


def _tc_head(h, yf, h1w, h1b, h2w, h2b):
    full = lambda: (0, 0)
    return pl.pallas_call(
        _head_body,
        in_specs=[
            pl.BlockSpec((N, H), full),
            pl.BlockSpec((N, 1), full),
            pl.BlockSpec((H, H), full),
            pl.BlockSpec((1, H), full),
            pl.BlockSpec((1, H), full),
            pl.BlockSpec((1, 1), full),
        ],
        out_specs=[
            pl.BlockSpec((N, 1), full),
            pl.BlockSpec((N, 1), full),
            pl.BlockSpec((1, 1), full),
        ],
        out_shape=[
            jax.ShapeDtypeStruct((N, 1), jnp.float32),
            jax.ShapeDtypeStruct((N, 1), jnp.float32),
            jax.ShapeDtypeStruct((1, 1), jnp.float32),
        ],
    )(h, yf, h1w, h1b, h2w, h2b)


def _combine_body(bce_r, fp_r, loss_r):
    loss_r[...] = bce_r[...] + 50.0 * (jnp.sum(fp_r[...]) / float(E))


def _tc_combine(bce, fparts):
    full = lambda: (0, 0)
    return pl.pallas_call(
        _combine_body,
        in_specs=[
            pl.BlockSpec((1, 1), full),
            pl.BlockSpec((32, 16), full),
        ],
        out_specs=pl.BlockSpec((1, 1), full),
        out_shape=jax.ShapeDtypeStruct((1, 1), jnp.float32),
    )(bce, fparts)


_MESH = plsc.VectorSubcoreMesh(core_axis_name="c", subcore_axis_name="s")


@functools.partial(
    pl.kernel,
    mesh=_MESH,
    out_type=jax.ShapeDtypeStruct((2, NPAD, HH), jnp.float32),
    compiler_params=pltpu.CompilerParams(use_tc_tiling_on_sc=False),
    scratch_types=[
        pltpu.VMEM((SCHUNKS, CHUNK), jnp.int32),
        pltpu.VMEM((SCHUNKS, CHUNK), jnp.int32),
        pltpu.VMEM((CHUNK, HH), jnp.float32),
        pltpu.VMEM((CHUNK, HH), jnp.float32),
        pltpu.VMEM_SHARED((NPAD, HH), jnp.float32),
        pltpu.SemaphoreType.DMA,
        pltpu.SemaphoreType.DMA,
    ],
)
def _sc_segsum(h2_hbm, ei_hbm, z_hbm, out_hbm, src_v, dst_v, buf0, buf1,
               agg_sh, sem0, sem1):
    c = lax.axis_index("c")
    s = lax.axis_index("s")
    base = s * ROWS_PER_SUB
    pltpu.sync_copy(z_hbm.at[pl.ds(base, ROWS_PER_SUB)],
                    agg_sh.at[pl.ds(base, ROWS_PER_SUB)])
    pltpu.sync_copy(ei_hbm.at[0, s], src_v)
    pltpu.sync_copy(ei_hbm.at[1, s], dst_v)
    plsc.subcore_barrier()

    tab = h2_hbm.at[c]
    pltpu.async_copy(tab.at[src_v.at[0]], buf0, sem0)
    pltpu.async_copy(tab.at[src_v.at[1]], buf1, sem1)

    @pl.loop(0, SCHUNKS, step=2)
    def _(j):
        pltpu.make_async_copy(tab.at[src_v.at[j]], buf0, sem0).wait()
        pltpu.sync_copy(buf0, agg_sh.at[dst_v.at[j]], add=True)

        @pl.when(j + 2 < SCHUNKS)
        def _():
            pltpu.async_copy(tab.at[src_v.at[j + 2]], buf0, sem0)

        pltpu.make_async_copy(tab.at[src_v.at[j + 1]], buf1, sem1).wait()
        pltpu.sync_copy(buf1, agg_sh.at[dst_v.at[j + 1]], add=True)

        @pl.when(j + 3 < SCHUNKS)
        def _():
            pltpu.async_copy(tab.at[src_v.at[j + 3]], buf1, sem1)

    plsc.subcore_barrier()
    pltpu.sync_copy(agg_sh.at[pl.ds(base, ROWS_PER_SUB)],
                    out_hbm.at[c, pl.ds(base, ROWS_PER_SUB)])


@functools.partial(
    pl.kernel,
    mesh=_MESH,
    out_type=jax.ShapeDtypeStruct((32, 16), jnp.float32),
    compiler_params=pltpu.CompilerParams(needs_layout_passes=False),
    scratch_types=[
        pltpu.VMEM((N,), jnp.float32),
        pltpu.VMEM((EDGES_PER_SUB,), jnp.int32),
        pltpu.VMEM((EDGES_PER_SUB,), jnp.int32),
        pltpu.VMEM((16,), jnp.float32),
    ],
)
def _sc_feas(probs_hbm, ei_hbm, out_hbm, ptab, src_v, dst_v, acc_v):
    c = lax.axis_index("c")
    s = lax.axis_index("s")
    w = c * 16 + s
    pltpu.sync_copy(probs_hbm, ptab)
    pltpu.sync_copy(ei_hbm.at[0, w], src_v)
    pltpu.sync_copy(ei_hbm.at[1, w], dst_v)
    acc_v[...] = jnp.zeros((16,), jnp.float32)

    @pl.loop(0, EDGES_PER_SUB, step=16)
    def _(i):
        si = src_v[pl.ds(i, 16)]
        di = dst_v[pl.ds(i, 16)]
        a = plsc.load_gather(ptab, [si])
        b = plsc.load_gather(ptab, [di])
        acc_v[...] = acc_v[...] + a * b

    pltpu.sync_copy(acc_v, out_hbm.at[w])


def kernel(x, edge_index, y, h_prev, y_prev, step_count, params):
    p = params
    ei_seg = edge_index.reshape(2, 16, SCHUNKS, CHUNK)
    ei_feas = edge_index.reshape(2, 32, EDGES_PER_SUB)
    zeros_pad = jnp.zeros((NPAD, HH), jnp.float32)

    ipa = p['ip_W'][:H]
    ipb = p['ip_W'][H:2 * H]
    ipc = p['ip_W'][2 * H:2 * H + 1]
    h, h2 = _tc_input(
        x, h_prev, y_prev,
        p['xe_W'], p['xe_b'].reshape(1, H),
        p['xn_g'].reshape(1, H), p['xn_b'].reshape(1, H),
        ipa, ipb, ipc, p['ip_b'].reshape(1, H),
        p['in_g'].reshape(1, H), p['in_b'].reshape(1, H))

    for lp in p['layers']:
        parts = _sc_segsum(h2, ei_seg, zeros_pad)
        h, h2 = _tc_layer(
            h, parts, lp['eps'].reshape(1, 1),
            lp['m1_W'], lp['m1_b'].reshape(1, 2 * H),
            lp['mn_g'].reshape(1, 2 * H), lp['mn_b'].reshape(1, 2 * H),
            lp['m2_W'], lp['m2_b'].reshape(1, H),
            lp['n_g'].reshape(1, H), lp['n_b'].reshape(1, H))

    yf = y.astype(jnp.float32).reshape(N, 1)
    logits, probs2d, bce = _tc_head(
        h, yf, p['h1_W'], p['h1_b'].reshape(1, H),
        p['h2_W'].reshape(1, H), p['h2_b'].reshape(1, 1))

    fparts = _sc_feas(probs2d.reshape(N), ei_feas)
    loss = _tc_combine(bce, fparts).reshape(())

    return loss, logits, h, probs2d.reshape(N)


# speedup vs baseline: 14.6339x; 1.0872x over previous
import functools

import numpy as np
import jax
import jax.numpy as jnp
from jax import lax
from jax.experimental import pallas as pl
from jax.experimental.pallas import tpu as pltpu
from jax.experimental.pallas import tpu_sc as plsc

N = 10000
E = 320000
H = 128
NPAD = 10240
ROWS_PER_SUB = NPAD // 16
EDGES_PER_SUB = E // 32
HH = H // 2
EDGES_PER_TILE = E // 16
SCHUNKS = 160
CHUNK = 125

_INV_SQRT2 = float(1.0 / np.sqrt(2.0))

_DOT = functools.partial(
    jnp.dot, precision=lax.Precision.HIGHEST,
    preferred_element_type=jnp.float32)


def _ln(v, g, b):
    mu = jnp.mean(v, axis=-1, keepdims=True)
    var = jnp.mean((v - mu) ** 2, axis=-1, keepdims=True)
    return (v - mu) * lax.rsqrt(var + 1e-5) * g + b


def _gelu(v):
    return 0.5 * v * (1.0 + lax.erf(v * _INV_SQRT2))


def _input_body(x_r, hp_r, yp_r, xew_r, xeb_r, xng_r, xnb_r,
                ipa_r, ipb_r, ipc_r, ipbias_r, ing_r, inb_r, o_r, o2_r):
    x_emb = _ln(_DOT(x_r[...], xew_r[...]) + xeb_r[...], xng_r[...], xnb_r[...])
    y_in = jax.nn.sigmoid(yp_r[...])
    pre = (_DOT(x_emb, ipa_r[...]) + _DOT(hp_r[...], ipb_r[...])
           + y_in * ipc_r[...] + ipbias_r[...])
    res = _ln(pre, ing_r[...], inb_r[...])
    o_r[...] = res
    o2_r[...] = jnp.stack((res[:, :HH], res[:, HH:]), axis=0)


def _tc_input(x, h_prev, y_prev, xew, xeb, xng, xnb, ipa, ipb, ipc, ipbias,
              ing, inb):
    blk = 2000
    grid = (N // blk,)
    row = lambda i: (i, 0)
    full = lambda i: (0, 0)
    return pl.pallas_call(
        _input_body,
        grid=grid,
        in_specs=[
            pl.BlockSpec((blk, H), row),
            pl.BlockSpec((blk, H), row),
            pl.BlockSpec((blk, 1), row),
            pl.BlockSpec((H, H), full),
            pl.BlockSpec((1, H), full),
            pl.BlockSpec((1, H), full),
            pl.BlockSpec((1, H), full),
            pl.BlockSpec((H, H), full),
            pl.BlockSpec((H, H), full),
            pl.BlockSpec((1, H), full),
            pl.BlockSpec((1, H), full),
            pl.BlockSpec((1, H), full),
            pl.BlockSpec((1, H), full),
        ],
        out_specs=[
            pl.BlockSpec((blk, H), row),
            pl.BlockSpec((2, blk, HH), lambda i: (0, i, 0)),
        ],
        out_shape=[
            jax.ShapeDtypeStruct((N, H), jnp.float32),
            jax.ShapeDtypeStruct((2, N, HH), jnp.float32),
        ],
    )(x, h_prev, y_prev, xew, xeb, xng, xnb, ipa, ipb, ipc, ipbias, ing, inb)


def _layer_body(h_r, p_r, eps_r, m1w_r, m1b_r, mng_r, mnb_r,
                m2w_r, m2b_r, ng_r, nb_r, o_r, o2_r):
    h = h_r[...]
    parts = p_r[...]
    agg = jnp.concatenate((parts[0], parts[1]), axis=-1)
    zc = (1.0 + eps_r[...]) * h + agg
    m = _DOT(zc, m1w_r[...]) + m1b_r[...]
    m = _gelu(_ln(m, mng_r[...], mnb_r[...]))
    m = _DOT(m, m2w_r[...]) + m2b_r[...]
    res = _ln(h + _gelu(m), ng_r[...], nb_r[...])
    o_r[...] = res
    o2_r[...] = jnp.stack((res[:, :HH], res[:, HH:]), axis=0)


def _tc_layer(h, parts, eps, m1w, m1b, mng, mnb, m2w, m2b, ng, nb):
    blk = 2000
    grid = (N // blk,)
    row = lambda i: (i, 0)
    full = lambda i: (0, 0)
    return pl.pallas_call(
        _layer_body,
        grid=grid,
        in_specs=[
            pl.BlockSpec((blk, H), row),
            pl.BlockSpec((2, blk, HH), lambda i: (0, i, 0)),
            pl.BlockSpec((1, 1), full),
            pl.BlockSpec((H, 2 * H), full),
            pl.BlockSpec((1, 2 * H), full),
            pl.BlockSpec((1, 2 * H), full),
            pl.BlockSpec((1, 2 * H), full),
            pl.BlockSpec((2 * H, H), full),
            pl.BlockSpec((1, H), full),
            pl.BlockSpec((1, H), full),
            pl.BlockSpec((1, H), full),
        ],
        out_specs=[
            pl.BlockSpec((blk, H), row),
            pl.BlockSpec((2, blk, HH), lambda i: (0, i, 0)),
        ],
        out_shape=[
            jax.ShapeDtypeStruct((N, H), jnp.float32),
            jax.ShapeDtypeStruct((2, N, HH), jnp.float32),
        ],
    )(h, parts, eps, m1w, m1b, mng, mnb, m2w, m2b, ng, nb)


def _head_body(h_r, yf_r, h1w_r, h1b_r, h2w_r, h2b_r,
               logits_r, probs_r, bce_r):
    t = _gelu(_DOT(h_r[...], h1w_r[...]) + h1b_r[...])
    lg = jnp.sum(t * h2w_r[...], axis=-1, keepdims=True) + h2b_r[...]
    logits_r[...] = lg
    z = jnp.clip(lg, -10.0, 10.0)
    probs_r[...] = jax.nn.sigmoid(z)
    labels = yf_r[...]
    pos = jnp.clip(jnp.sum(labels), 1.0, None)
    neg = jnp.clip(float(N) - pos, 1.0, None)
    pw = neg / pos
    ls_pos = -jnp.log1p(jnp.exp(-z))
    ls_neg = -jnp.log1p(jnp.exp(z))
    terms = -(pw * labels * ls_pos + (1.0 - labels) * ls_neg)
    bce_r[...] = jnp.reshape(jnp.mean(terms), (1, 1))


def _tc_head(h, yf, h1w, h1b, h2w, h2b):
    full = lambda: (0, 0)
    return pl.pallas_call(
        _head_body,
        in_specs=[
            pl.BlockSpec((N, H), full),
            pl.BlockSpec((N, 1), full),
            pl.BlockSpec((H, H), full),
            pl.BlockSpec((1, H), full),
            pl.BlockSpec((1, H), full),
            pl.BlockSpec((1, 1), full),
        ],
        out_specs=[
            pl.BlockSpec((N, 1), full),
            pl.BlockSpec((N, 1), full),
            pl.BlockSpec((1, 1), full),
        ],
        out_shape=[
            jax.ShapeDtypeStruct((N, 1), jnp.float32),
            jax.ShapeDtypeStruct((N, 1), jnp.float32),
            jax.ShapeDtypeStruct((1, 1), jnp.float32),
        ],
    )(h, yf, h1w, h1b, h2w, h2b)


def _combine_body(bce_r, fp_r, loss_r):
    loss_r[...] = bce_r[...] + 50.0 * (jnp.sum(fp_r[...]) / float(E))


def _tc_combine(bce, fparts):
    full = lambda: (0, 0)
    return pl.pallas_call(
        _combine_body,
        in_specs=[
            pl.BlockSpec((1, 1), full),
            pl.BlockSpec((32, 16), full),
        ],
        out_specs=pl.BlockSpec((1, 1), full),
        out_shape=jax.ShapeDtypeStruct((1, 1), jnp.float32),
    )(bce, fparts)


_MESH = plsc.VectorSubcoreMesh(core_axis_name="c", subcore_axis_name="s")


@functools.partial(
    pl.kernel,
    mesh=_MESH,
    out_type=jax.ShapeDtypeStruct((2, NPAD, HH), jnp.float32),
    compiler_params=pltpu.CompilerParams(use_tc_tiling_on_sc=False),
    scratch_types=[
        pltpu.VMEM((SCHUNKS, CHUNK), jnp.int32),
        pltpu.VMEM((SCHUNKS, CHUNK), jnp.int32),
        pltpu.VMEM((CHUNK, HH), jnp.float32),
        pltpu.VMEM((CHUNK, HH), jnp.float32),
        pltpu.VMEM((CHUNK, HH), jnp.float32),
        pltpu.VMEM((CHUNK, HH), jnp.float32),
        pltpu.VMEM_SHARED((NPAD, HH), jnp.float32),
        pltpu.SemaphoreType.DMA,
        pltpu.SemaphoreType.DMA,
        pltpu.SemaphoreType.DMA,
        pltpu.SemaphoreType.DMA,
        pltpu.SemaphoreType.DMA,
        pltpu.SemaphoreType.DMA,
        pltpu.SemaphoreType.DMA,
        pltpu.SemaphoreType.DMA,
    ],
)
def _sc_segsum(h2_hbm, ei_hbm, z_hbm, out_hbm, src_v, dst_v,
               buf0, buf1, buf2, buf3, agg_sh,
               g0, g1, g2, g3, s0, s1, s2, s3):
    c = lax.axis_index("c")
    s = lax.axis_index("s")
    base = s * ROWS_PER_SUB
    pltpu.sync_copy(z_hbm.at[pl.ds(base, ROWS_PER_SUB)],
                    agg_sh.at[pl.ds(base, ROWS_PER_SUB)])
    pltpu.sync_copy(ei_hbm.at[0, s], src_v)
    pltpu.sync_copy(ei_hbm.at[1, s], dst_v)
    plsc.subcore_barrier()

    tab = h2_hbm.at[c]
    bufs = (buf0, buf1, buf2, buf3)
    gsems = (g0, g1, g2, g3)
    ssems = (s0, s1, s2, s3)

    for k in range(4):
        pltpu.async_copy(tab.at[src_v.at[k]], bufs[k], gsems[k])

    @pl.loop(0, SCHUNKS, step=4)
    def _(j):
        for k in range(4):
            pltpu.make_async_copy(tab.at[src_v.at[j + k]], bufs[k],
                                  gsems[k]).wait()
            pltpu.async_copy(bufs[k], agg_sh.at[dst_v.at[j + k]], ssems[k],
                             add=True)
        for k in range(4):
            pltpu.make_async_copy(bufs[k], agg_sh.at[dst_v.at[j + k]],
                                  ssems[k]).wait()

            @pl.when(j + k + 4 < SCHUNKS)
            def _():
                pltpu.async_copy(tab.at[src_v.at[j + k + 4]], bufs[k],
                                 gsems[k])

    plsc.subcore_barrier()
    pltpu.sync_copy(agg_sh.at[pl.ds(base, ROWS_PER_SUB)],
                    out_hbm.at[c, pl.ds(base, ROWS_PER_SUB)])


@functools.partial(
    pl.kernel,
    mesh=_MESH,
    out_type=jax.ShapeDtypeStruct((32, 16), jnp.float32),
    compiler_params=pltpu.CompilerParams(needs_layout_passes=False),
    scratch_types=[
        pltpu.VMEM((N,), jnp.float32),
        pltpu.VMEM((EDGES_PER_SUB,), jnp.int32),
        pltpu.VMEM((EDGES_PER_SUB,), jnp.int32),
        pltpu.VMEM((16,), jnp.float32),
    ],
)
def _sc_feas(probs_hbm, ei_hbm, out_hbm, ptab, src_v, dst_v, acc_v):
    c = lax.axis_index("c")
    s = lax.axis_index("s")
    w = c * 16 + s
    pltpu.sync_copy(probs_hbm, ptab)
    pltpu.sync_copy(ei_hbm.at[0, w], src_v)
    pltpu.sync_copy(ei_hbm.at[1, w], dst_v)
    acc_v[...] = jnp.zeros((16,), jnp.float32)

    @pl.loop(0, EDGES_PER_SUB, step=16)
    def _(i):
        si = src_v[pl.ds(i, 16)]
        di = dst_v[pl.ds(i, 16)]
        a = plsc.load_gather(ptab, [si])
        b = plsc.load_gather(ptab, [di])
        acc_v[...] = acc_v[...] + a * b

    pltpu.sync_copy(acc_v, out_hbm.at[w])


def kernel(x, edge_index, y, h_prev, y_prev, step_count, params):
    p = params
    ei_seg = edge_index.reshape(2, 16, SCHUNKS, CHUNK)
    ei_feas = edge_index.reshape(2, 32, EDGES_PER_SUB)
    zeros_pad = jnp.zeros((NPAD, HH), jnp.float32)

    ipa = p['ip_W'][:H]
    ipb = p['ip_W'][H:2 * H]
    ipc = p['ip_W'][2 * H:2 * H + 1]
    h, h2 = _tc_input(
        x, h_prev, y_prev,
        p['xe_W'], p['xe_b'].reshape(1, H),
        p['xn_g'].reshape(1, H), p['xn_b'].reshape(1, H),
        ipa, ipb, ipc, p['ip_b'].reshape(1, H),
        p['in_g'].reshape(1, H), p['in_b'].reshape(1, H))

    for lp in p['layers']:
        parts = _sc_segsum(h2, ei_seg, zeros_pad)
        h, h2 = _tc_layer(
            h, parts, lp['eps'].reshape(1, 1),
            lp['m1_W'], lp['m1_b'].reshape(1, 2 * H),
            lp['mn_g'].reshape(1, 2 * H), lp['mn_b'].reshape(1, 2 * H),
            lp['m2_W'], lp['m2_b'].reshape(1, H),
            lp['n_g'].reshape(1, H), lp['n_b'].reshape(1, H))

    yf = y.astype(jnp.float32).reshape(N, 1)
    logits, probs2d, bce = _tc_head(
        h, yf, p['h1_W'], p['h1_b'].reshape(1, H),
        p['h2_W'].reshape(1, H), p['h2_b'].reshape(1, 1))

    fparts = _sc_feas(probs2d.reshape(N), ei_feas)
    loss = _tc_combine(bce, fparts).reshape(())

    return loss, logits, h, probs2d.reshape(N)
